# Initial kernel scaffold; baseline (speedup 1.0000x reference)
import jax
import jax.numpy as jnp
from jax.experimental import pallas as pl


def kernel(batch_img, batch_u, conv_wt, conv_b, bk_fc_w, bk_fc_b, fn_w1, fn_b1, fn_w2, fn_b2, gcn_w1, gcn_b1, gcn_w2, gcn_b2, gcn_w3, gcn_b3, fc_w1a, fc_b1, fc_w1b, fc_w2, fc_b2):
    raise NotImplementedError("write your pallas kernel here")


# speedup vs baseline: 3.2409x; 3.2409x over previous
import functools

import jax
import jax.numpy as jnp
from jax import lax
from jax.experimental import pallas as pl
from jax.experimental.pallas import tpu as pltpu


def _round_up(x, m):
    return ((x + m - 1) // m) * m


def _pick_lane_tile(n, cap):
    best = None
    t = 128
    while t <= min(n, cap):
        if n % t == 0:
            best = t
        t += 128
    return best if best is not None else n


def _backbone_kernel(x_ref, cwt_ref, cb_ref, fw_ref, fb_ref, o_ref, pool_ref,
                     *, inv_hw):
    hw = pl.program_id(1)

    @pl.when(hw == 0)
    def _():
        pool_ref[...] = jnp.zeros_like(pool_ref)

    x = x_ref[0].astype(jnp.bfloat16)
    h = jnp.dot(cwt_ref[...], x, preferred_element_type=jnp.float32)
    h = jnp.maximum(h + cb_ref[...], 0.0)
    pool_ref[...] += jnp.sum(h, axis=1, keepdims=True)

    @pl.when(hw == pl.num_programs(1) - 1)
    def _():
        pooled = pool_ref[...] * inv_hw
        feat = jnp.sum(pooled * fw_ref[...], axis=0, keepdims=True) + fb_ref[...]
        o_ref[0] = jnp.maximum(feat, 0.0).astype(jnp.bfloat16)


def _backbone_forward(x_bc_hw, conv_wt, conv_b, bk_fc_w, bk_fc_b):
    B, C, HW = x_bc_hw.shape
    conv_mid = conv_wt.shape[0]
    feat_dim = bk_fc_w.shape[1]
    t_hw = _pick_lane_tile(HW, 16384)
    n_hw = HW // t_hw

    flops = 2 * B * HW * C * conv_mid + 2 * B * conv_mid * feat_dim
    bytes_acc = (B * C * HW * 4 + C * conv_mid * 2 + conv_mid * feat_dim * 4
                 + B * feat_dim * 2)
    out = pl.pallas_call(
        functools.partial(_backbone_kernel, inv_hw=1.0 / float(HW)),
        out_shape=jax.ShapeDtypeStruct((B, 1, feat_dim), jnp.bfloat16),
        grid_spec=pltpu.PrefetchScalarGridSpec(
            num_scalar_prefetch=0,
            grid=(B, n_hw),
            in_specs=[
                pl.BlockSpec((1, C, t_hw), lambda b, h: (b, 0, h)),
                pl.BlockSpec((conv_mid, C), lambda b, h: (0, 0)),
                pl.BlockSpec((conv_mid, 1), lambda b, h: (0, 0)),
                pl.BlockSpec((conv_mid, feat_dim), lambda b, h: (0, 0)),
                pl.BlockSpec((1, feat_dim), lambda b, h: (0, 0)),
            ],
            out_specs=pl.BlockSpec((1, 1, feat_dim), lambda b, h: (b, 0, 0)),
            scratch_shapes=[pltpu.VMEM((conv_mid, 1), jnp.float32)],
        ),
        compiler_params=pltpu.CompilerParams(
            dimension_semantics=("parallel", "arbitrary")),
        cost_estimate=pl.CostEstimate(flops=flops, transcendentals=0,
                                      bytes_accessed=bytes_acc),
    )(x_bc_hw, conv_wt, conv_b, bk_fc_w, bk_fc_b)
    return out.reshape(B, feat_dim)


def _gcn_kernel(u_ref, fnw1_ref, fnb1_ref, fnw2_ref, fnb2_ref,
                a_ref, w1_ref, b1_ref, w2_ref, b2_ref, w3_ref, b3_ref,
                r_ref, o_ref, *, num_graphs, node_size):
    GN = num_graphs * node_size

    u = u_ref[...].astype(jnp.bfloat16)
    t = jnp.dot(u, fnw1_ref[...], preferred_element_type=jnp.float32) + fnb1_ref[...]
    t = jnp.maximum(t, 0.0).astype(jnp.bfloat16)
    h0 = jnp.dot(t, fnw2_ref[...], preferred_element_type=jnp.float32) + fnb2_ref[...]

    A = a_ref[...]

    h0b = h0.astype(jnp.bfloat16)
    tiled = jnp.concatenate([h0b] * num_graphs, axis=1)
    row_g = lax.broadcasted_iota(jnp.int32, (GN, GN), 0) // node_size
    col_g = lax.broadcasted_iota(jnp.int32, (GN, GN), 1) // node_size
    ht = jnp.where(row_g == col_g, tiled, jnp.bfloat16(0))
    z = jnp.dot(ht, w1_ref[...], preferred_element_type=jnp.float32)
    z = jnp.dot(A, z.astype(jnp.bfloat16),
                preferred_element_type=jnp.float32) + b1_ref[...]
    h1 = jnp.maximum(z, 0.0).astype(jnp.bfloat16)

    z = jnp.dot(h1, w2_ref[...], preferred_element_type=jnp.float32)
    z = jnp.dot(A, z.astype(jnp.bfloat16),
                preferred_element_type=jnp.float32) + b2_ref[...]
    h2 = jnp.maximum(z, 0.0).astype(jnp.bfloat16)

    z = jnp.dot(h2, w3_ref[...], preferred_element_type=jnp.float32)
    z = jnp.dot(A, z.astype(jnp.bfloat16),
                preferred_element_type=jnp.float32) + b3_ref[...]
    o_ref[...] = jnp.dot(r_ref[...], z.astype(jnp.bfloat16),
                         preferred_element_type=jnp.float32).astype(jnp.bfloat16)


def _gcn_forward(batch_u, A_bd, R, fn_w1, fn_b1, fn_w2, fn_b2,
                 gcn_w1, gcn_b1, gcn_w2, gcn_b2, gcn_w3, gcn_b3):
    G, Nn, Fin = batch_u.shape
    GN = G * Nn
    H256 = fn_w1.shape[1]
    ns = fn_w2.shape[1]
    H1 = gcn_w1.shape[1]
    H2 = gcn_w2.shape[1]
    Fo = gcn_w3.shape[1]
    u2d = batch_u.reshape(GN, Fin)

    flops = (2 * GN * (Fin * H256 + H256 * ns) + 2 * GN * GN * H1
             + 2 * GN * GN * (H1 + H2 + Fo)
             + 2 * GN * (H1 * H2 + H2 * Fo) + 2 * G * GN * Fo)
    bytes_acc = (GN * Fin * 4 + Fin * H256 * 2 + H256 * ns * 2 + GN * H1 * 2
                 + H1 * H2 * 2 + H2 * Fo * 2 + GN * GN * 2 + G * GN * 2
                 + G * Fo * 2)

    return pl.pallas_call(
        functools.partial(_gcn_kernel, num_graphs=G, node_size=Nn),
        out_shape=jax.ShapeDtypeStruct((G, Fo), jnp.bfloat16),
        grid_spec=pltpu.PrefetchScalarGridSpec(
            num_scalar_prefetch=0,
            grid=(1,),
            in_specs=[
                pl.BlockSpec((GN, Fin), lambda i: (0, 0)),
                pl.BlockSpec((Fin, H256), lambda i: (0, 0)),
                pl.BlockSpec((1, H256), lambda i: (0, 0)),
                pl.BlockSpec((H256, ns), lambda i: (0, 0)),
                pl.BlockSpec((1, ns), lambda i: (0, 0)),
                pl.BlockSpec((GN, GN), lambda i: (0, 0)),
                pl.BlockSpec((GN, H1), lambda i: (0, 0)),
                pl.BlockSpec((1, H1), lambda i: (0, 0)),
                pl.BlockSpec((H1, H2), lambda i: (0, 0)),
                pl.BlockSpec((1, H2), lambda i: (0, 0)),
                pl.BlockSpec((H2, Fo), lambda i: (0, 0)),
                pl.BlockSpec((1, Fo), lambda i: (0, 0)),
                pl.BlockSpec((G, GN), lambda i: (0, 0)),
            ],
            out_specs=pl.BlockSpec((G, Fo), lambda i: (0, 0)),
        ),
        compiler_params=pltpu.CompilerParams(
            dimension_semantics=("arbitrary",)),
        cost_estimate=pl.CostEstimate(flops=flops, transcendentals=0,
                                      bytes_accessed=bytes_acc),
    )(u2d, fn_w1, fn_b1, fn_w2, fn_b2, A_bd, gcn_w1, gcn_b1,
      gcn_w2, gcn_b2, gcn_w3, gcn_b3, R)


def _classifier_kernel(f_ref, fg_ref, w1a_ref, w1b_ref, b1_ref, w2_ref, b2_ref,
                       o_ref, f_sc, fg_sc):
    j = pl.program_id(0)

    @pl.when(j == 0)
    def _():
        o_ref[...] = jnp.zeros_like(o_ref)
        f_sc[...] = jnp.maximum(f_ref[...], 0.0).astype(jnp.bfloat16)
        fg_sc[...] = jnp.maximum(fg_ref[...], 0.0).astype(jnp.bfloat16)

    h = jnp.dot(f_sc[...], w1a_ref[...], preferred_element_type=jnp.float32)
    h = h + jnp.dot(fg_sc[...], w1b_ref[...], preferred_element_type=jnp.float32)
    h = jnp.maximum(h + b1_ref[...], 0.0).astype(jnp.bfloat16)
    o_ref[...] += jnp.dot(h, w2_ref[...], preferred_element_type=jnp.float32)

    @pl.when(j == pl.num_programs(0) - 1)
    def _():
        o_ref[...] = o_ref[...] + b2_ref[...]


def _classifier_forward(feature, feature_g, fc_w1a, fc_b1, fc_w1b, fc_w2, fc_b2):
    B, F1 = feature.shape
    G, F2 = feature_g.shape
    H1 = fc_w1a.shape[1]
    num_classes = fc_w2.shape[1]

    t_n1 = _pick_lane_tile(H1, 512)
    n1 = H1 // t_n1
    Np = _round_up(max(num_classes, 128), 128)
    w2p = jnp.pad(fc_w2, ((0, 0), (0, Np - num_classes)))
    b2p = jnp.pad(fc_b2, ((0, 0), (0, Np - num_classes)))

    flops = 2 * B * (F1 * H1 + F2 * H1 + H1 * Np)
    bytes_acc = (F1 * H1 * 2 + F2 * H1 * 2 + H1 * Np * 2
                 + B * (F1 + F2) * 2 + B * Np * 4)
    out = pl.pallas_call(
        _classifier_kernel,
        out_shape=jax.ShapeDtypeStruct((B, Np), jnp.float32),
        grid_spec=pltpu.PrefetchScalarGridSpec(
            num_scalar_prefetch=0,
            grid=(n1,),
            in_specs=[
                pl.BlockSpec((B, F1), lambda j: (0, 0)),
                pl.BlockSpec((B, F2), lambda j: (0, 0)),
                pl.BlockSpec((F1, t_n1), lambda j: (0, j)),
                pl.BlockSpec((F2, t_n1), lambda j: (0, j)),
                pl.BlockSpec((1, t_n1), lambda j: (0, j)),
                pl.BlockSpec((t_n1, Np), lambda j: (j, 0)),
                pl.BlockSpec((1, Np), lambda j: (0, 0)),
            ],
            out_specs=pl.BlockSpec((B, Np), lambda j: (0, 0)),
            scratch_shapes=[pltpu.VMEM((B, F1), jnp.bfloat16),
                            pltpu.VMEM((B, F2), jnp.bfloat16)],
        ),
        compiler_params=pltpu.CompilerParams(
            dimension_semantics=("arbitrary",),
            vmem_limit_bytes=64 * 1024 * 1024),
        cost_estimate=pl.CostEstimate(flops=flops, transcendentals=0,
                                      bytes_accessed=bytes_acc),
    )(feature, feature_g, fc_w1a, fc_w1b, fc_b1, w2p, b2p)
    return out[:, :num_classes]


def _build_norm_adj(node_num):
    idx = jnp.arange(node_num)
    A = jnp.zeros((node_num, node_num), jnp.float32)
    A = A.at[idx, (idx + 1) % node_num].set(1.0)
    A = A.at[(idx + 1) % node_num, idx].set(1.0)
    A = A + jnp.eye(node_num, dtype=jnp.float32)
    dinv = 1.0 / jnp.sqrt(A.sum(axis=1))
    return A * dinv[:, None] * dinv[None, :]


def kernel(batch_img, batch_u, conv_wt, conv_b, bk_fc_w, bk_fc_b,
           fn_w1, fn_b1, fn_w2, fn_b2, gcn_w1, gcn_b1, gcn_w2, gcn_b2,
           gcn_w3, gcn_b3, fc_w1a, fc_b1, fc_w1b, fc_w2, fc_b2):
    B, C, H, W = batch_img.shape
    HW = H * W
    G, node_num, _ = batch_u.shape

    feature = _backbone_forward(batch_img.reshape(B, C, HW),
                                conv_wt, conv_b, bk_fc_w, bk_fc_b)

    A_hat = _build_norm_adj(node_num)
    A_bd = jnp.kron(jnp.eye(G, dtype=jnp.float32), A_hat).astype(jnp.bfloat16)
    R = jnp.kron(jnp.eye(G, dtype=jnp.float32),
                 jnp.full((1, node_num), 1.0 / node_num, jnp.float32)
                 ).astype(jnp.bfloat16)

    feature_g = _gcn_forward(batch_u, A_bd, R, fn_w1, fn_b1, fn_w2, fn_b2,
                             gcn_w1, gcn_b1, gcn_w2, gcn_b2, gcn_w3, gcn_b3)

    return _classifier_forward(feature, feature_g,
                               fc_w1a, fc_b1, fc_w1b, fc_w2, fc_b2)


# speedup vs baseline: 4.2404x; 1.3084x over previous
import functools

import jax
import jax.numpy as jnp
from jax import lax
from jax.experimental import pallas as pl
from jax.experimental.pallas import tpu as pltpu


def _round_up(x, m):
    return ((x + m - 1) // m) * m


def _pick_lane_tile(n, cap):
    best = None
    t = 128
    while t <= min(n, cap):
        if n % t == 0:
            best = t
        t += 128
    return best if best is not None else n


def _backbone_kernel(x_ref, cwt_ref, cb_ref, fw_ref, fb_ref, o_ref, *, inv_hw):
    x = x_ref[0].astype(jnp.bfloat16)
    h = jnp.dot(cwt_ref[...], x,
                preferred_element_type=jnp.float32).astype(jnp.bfloat16)
    h = jnp.maximum(h + cb_ref[...].astype(jnp.bfloat16), jnp.bfloat16(0))
    pooled = jnp.sum(h, axis=1, keepdims=True).astype(jnp.float32) * inv_hw
    feat = jnp.sum(pooled * fw_ref[...], axis=0, keepdims=True) + fb_ref[...]
    o_ref[0] = jnp.maximum(feat, 0.0).astype(jnp.bfloat16)


def _backbone_forward(x_bc_hw, conv_wt, conv_b, bk_fc_w, bk_fc_b):
    B, C, HW = x_bc_hw.shape
    conv_mid = conv_wt.shape[0]
    feat_dim = bk_fc_w.shape[1]

    flops = 2 * B * HW * C * conv_mid + 2 * B * conv_mid * feat_dim
    bytes_acc = (B * C * HW * 4 + C * conv_mid * 2 + conv_mid * feat_dim * 4
                 + B * feat_dim * 2)
    out = pl.pallas_call(
        functools.partial(_backbone_kernel, inv_hw=1.0 / float(HW)),
        out_shape=jax.ShapeDtypeStruct((B, 1, feat_dim), jnp.bfloat16),
        grid_spec=pltpu.PrefetchScalarGridSpec(
            num_scalar_prefetch=0,
            grid=(B,),
            in_specs=[
                pl.BlockSpec((1, C, HW), lambda b: (b, 0, 0)),
                pl.BlockSpec((conv_mid, C), lambda b: (0, 0)),
                pl.BlockSpec((conv_mid, 1), lambda b: (0, 0)),
                pl.BlockSpec((conv_mid, feat_dim), lambda b: (0, 0)),
                pl.BlockSpec((1, feat_dim), lambda b: (0, 0)),
            ],
            out_specs=pl.BlockSpec((1, 1, feat_dim), lambda b: (b, 0, 0)),
        ),
        compiler_params=pltpu.CompilerParams(
            dimension_semantics=("parallel",),
            vmem_limit_bytes=64 * 1024 * 1024),
        cost_estimate=pl.CostEstimate(flops=flops, transcendentals=0,
                                      bytes_accessed=bytes_acc),
    )(x_bc_hw, conv_wt, conv_b, bk_fc_w, bk_fc_b)
    return out.reshape(B, feat_dim)


def _gcn_kernel(u_ref, fnw1_ref, fnb1_ref, fnw2_ref, fnb2_ref,
                a_ref, w1_ref, b1_ref, w2_ref, b2_ref, w3_ref, b3_ref,
                r_ref, o_ref, *, num_graphs, node_size):
    GN = num_graphs * node_size

    u = u_ref[...].astype(jnp.bfloat16)
    t = jnp.dot(u, fnw1_ref[...], preferred_element_type=jnp.float32) + fnb1_ref[...]
    t = jnp.maximum(t, 0.0).astype(jnp.bfloat16)
    h0 = jnp.dot(t, fnw2_ref[...], preferred_element_type=jnp.float32) + fnb2_ref[...]

    A = a_ref[...]

    h0b = h0.astype(jnp.bfloat16)
    tiled = jnp.concatenate([h0b] * num_graphs, axis=1)
    row_g = lax.broadcasted_iota(jnp.int32, (GN, GN), 0) // node_size
    col_g = lax.broadcasted_iota(jnp.int32, (GN, GN), 1) // node_size
    ht = jnp.where(row_g == col_g, tiled, jnp.bfloat16(0))
    z = jnp.dot(ht, w1_ref[...], preferred_element_type=jnp.float32)
    z = jnp.dot(A, z.astype(jnp.bfloat16),
                preferred_element_type=jnp.float32) + b1_ref[...]
    h1 = jnp.maximum(z, 0.0).astype(jnp.bfloat16)

    z = jnp.dot(h1, w2_ref[...], preferred_element_type=jnp.float32)
    z = jnp.dot(A, z.astype(jnp.bfloat16),
                preferred_element_type=jnp.float32) + b2_ref[...]
    h2 = jnp.maximum(z, 0.0).astype(jnp.bfloat16)

    z = jnp.dot(h2, w3_ref[...], preferred_element_type=jnp.float32)
    z = jnp.dot(A, z.astype(jnp.bfloat16),
                preferred_element_type=jnp.float32) + b3_ref[...]
    o_ref[...] = jnp.dot(r_ref[...], z.astype(jnp.bfloat16),
                         preferred_element_type=jnp.float32).astype(jnp.bfloat16)


def _gcn_forward(batch_u, A_bd, R, fn_w1, fn_b1, fn_w2, fn_b2,
                 gcn_w1, gcn_b1, gcn_w2, gcn_b2, gcn_w3, gcn_b3):
    G, Nn, Fin = batch_u.shape
    GN = G * Nn
    H256 = fn_w1.shape[1]
    ns = fn_w2.shape[1]
    H1 = gcn_w1.shape[1]
    H2 = gcn_w2.shape[1]
    Fo = gcn_w3.shape[1]
    u2d = batch_u.reshape(GN, Fin)

    flops = (2 * GN * (Fin * H256 + H256 * ns) + 2 * GN * GN * H1
             + 2 * GN * GN * (H1 + H2 + Fo)
             + 2 * GN * (H1 * H2 + H2 * Fo) + 2 * G * GN * Fo)
    bytes_acc = (GN * Fin * 4 + Fin * H256 * 2 + H256 * ns * 2 + GN * H1 * 2
                 + H1 * H2 * 2 + H2 * Fo * 2 + GN * GN * 2 + G * GN * 2
                 + G * Fo * 2)

    return pl.pallas_call(
        functools.partial(_gcn_kernel, num_graphs=G, node_size=Nn),
        out_shape=jax.ShapeDtypeStruct((G, Fo), jnp.bfloat16),
        grid_spec=pltpu.PrefetchScalarGridSpec(
            num_scalar_prefetch=0,
            grid=(1,),
            in_specs=[
                pl.BlockSpec((GN, Fin), lambda i: (0, 0)),
                pl.BlockSpec((Fin, H256), lambda i: (0, 0)),
                pl.BlockSpec((1, H256), lambda i: (0, 0)),
                pl.BlockSpec((H256, ns), lambda i: (0, 0)),
                pl.BlockSpec((1, ns), lambda i: (0, 0)),
                pl.BlockSpec((GN, GN), lambda i: (0, 0)),
                pl.BlockSpec((GN, H1), lambda i: (0, 0)),
                pl.BlockSpec((1, H1), lambda i: (0, 0)),
                pl.BlockSpec((H1, H2), lambda i: (0, 0)),
                pl.BlockSpec((1, H2), lambda i: (0, 0)),
                pl.BlockSpec((H2, Fo), lambda i: (0, 0)),
                pl.BlockSpec((1, Fo), lambda i: (0, 0)),
                pl.BlockSpec((G, GN), lambda i: (0, 0)),
            ],
            out_specs=pl.BlockSpec((G, Fo), lambda i: (0, 0)),
        ),
        compiler_params=pltpu.CompilerParams(
            dimension_semantics=("arbitrary",)),
        cost_estimate=pl.CostEstimate(flops=flops, transcendentals=0,
                                      bytes_accessed=bytes_acc),
    )(u2d, fn_w1, fn_b1, fn_w2, fn_b2, A_bd, gcn_w1, gcn_b1,
      gcn_w2, gcn_b2, gcn_w3, gcn_b3, R)


def _classifier_kernel(f_ref, fg_ref, w1a_ref, w1b_ref, b1_ref, w2_ref, b2_ref,
                       o_ref, f_sc, fg_sc):
    j = pl.program_id(0)

    @pl.when(j == 0)
    def _():
        o_ref[...] = jnp.zeros_like(o_ref)
        f_sc[...] = jnp.maximum(f_ref[...], 0.0).astype(jnp.bfloat16)
        fg_sc[...] = jnp.maximum(fg_ref[...], 0.0).astype(jnp.bfloat16)

    h = jnp.dot(f_sc[...], w1a_ref[...], preferred_element_type=jnp.float32)
    h = h + jnp.dot(fg_sc[...], w1b_ref[...], preferred_element_type=jnp.float32)
    h = jnp.maximum(h + b1_ref[...], 0.0).astype(jnp.bfloat16)
    o_ref[...] += jnp.dot(h, w2_ref[...], preferred_element_type=jnp.float32)

    @pl.when(j == pl.num_programs(0) - 1)
    def _():
        o_ref[...] = o_ref[...] + b2_ref[...]


def _classifier_forward(feature, feature_g, fc_w1a, fc_b1, fc_w1b, fc_w2, fc_b2):
    B, F1 = feature.shape
    G, F2 = feature_g.shape
    H1 = fc_w1a.shape[1]
    num_classes = fc_w2.shape[1]

    t_n1 = _pick_lane_tile(H1, 512)
    n1 = H1 // t_n1
    Np = _round_up(max(num_classes, 128), 128)
    w2p = jnp.pad(fc_w2, ((0, 0), (0, Np - num_classes)))
    b2p = jnp.pad(fc_b2, ((0, 0), (0, Np - num_classes)))

    flops = 2 * B * (F1 * H1 + F2 * H1 + H1 * Np)
    bytes_acc = (F1 * H1 * 2 + F2 * H1 * 2 + H1 * Np * 2
                 + B * (F1 + F2) * 2 + B * Np * 4)
    out = pl.pallas_call(
        _classifier_kernel,
        out_shape=jax.ShapeDtypeStruct((B, Np), jnp.float32),
        grid_spec=pltpu.PrefetchScalarGridSpec(
            num_scalar_prefetch=0,
            grid=(n1,),
            in_specs=[
                pl.BlockSpec((B, F1), lambda j: (0, 0)),
                pl.BlockSpec((B, F2), lambda j: (0, 0)),
                pl.BlockSpec((F1, t_n1), lambda j: (0, j)),
                pl.BlockSpec((F2, t_n1), lambda j: (0, j)),
                pl.BlockSpec((1, t_n1), lambda j: (0, j)),
                pl.BlockSpec((t_n1, Np), lambda j: (j, 0)),
                pl.BlockSpec((1, Np), lambda j: (0, 0)),
            ],
            out_specs=pl.BlockSpec((B, Np), lambda j: (0, 0)),
            scratch_shapes=[pltpu.VMEM((B, F1), jnp.bfloat16),
                            pltpu.VMEM((B, F2), jnp.bfloat16)],
        ),
        compiler_params=pltpu.CompilerParams(
            dimension_semantics=("arbitrary",),
            vmem_limit_bytes=64 * 1024 * 1024),
        cost_estimate=pl.CostEstimate(flops=flops, transcendentals=0,
                                      bytes_accessed=bytes_acc),
    )(feature, feature_g, fc_w1a, fc_w1b, fc_b1, w2p, b2p)
    return out[:, :num_classes]


def _build_norm_adj(node_num):
    idx = jnp.arange(node_num)
    A = jnp.zeros((node_num, node_num), jnp.float32)
    A = A.at[idx, (idx + 1) % node_num].set(1.0)
    A = A.at[(idx + 1) % node_num, idx].set(1.0)
    A = A + jnp.eye(node_num, dtype=jnp.float32)
    dinv = 1.0 / jnp.sqrt(A.sum(axis=1))
    return A * dinv[:, None] * dinv[None, :]


def kernel(batch_img, batch_u, conv_wt, conv_b, bk_fc_w, bk_fc_b,
           fn_w1, fn_b1, fn_w2, fn_b2, gcn_w1, gcn_b1, gcn_w2, gcn_b2,
           gcn_w3, gcn_b3, fc_w1a, fc_b1, fc_w1b, fc_w2, fc_b2):
    B, C, H, W = batch_img.shape
    HW = H * W
    G, node_num, _ = batch_u.shape

    feature = _backbone_forward(batch_img.reshape(B, C, HW),
                                conv_wt, conv_b, bk_fc_w, bk_fc_b)
    A_hat = _build_norm_adj(node_num)
    A_bd = jnp.kron(jnp.eye(G, dtype=jnp.float32), A_hat).astype(jnp.bfloat16)
    R = jnp.kron(jnp.eye(G, dtype=jnp.float32),
                 jnp.full((1, node_num), 1.0 / node_num, jnp.float32)
                 ).astype(jnp.bfloat16)

    feature_g = _gcn_forward(batch_u, A_bd, R, fn_w1, fn_b1, fn_w2, fn_b2,
                             gcn_w1, gcn_b1, gcn_w2, gcn_b2, gcn_w3, gcn_b3)

    return _classifier_forward(feature, feature_g,
                               fc_w1a, fc_b1, fc_w1b, fc_w2, fc_b2)


# speedup vs baseline: 4.5156x; 1.0649x over previous
import functools

import jax
import jax.numpy as jnp
from jax import lax
from jax.experimental import pallas as pl
from jax.experimental.pallas import tpu as pltpu


def _round_up(x, m):
    return ((x + m - 1) // m) * m


def _pick_lane_tile(n, cap):
    best = None
    t = 128
    while t <= min(n, cap):
        if n % t == 0:
            best = t
        t += 128
    return best if best is not None else n


def _backbone_kernel(x_ref, wbig_ref, cbig_ref, fw_ref, fb_ref, o_ref,
                     *, conv_mid, inv_hw):
    x = x_ref[0].astype(jnp.bfloat16)
    h = jnp.dot(wbig_ref[...], x, preferred_element_type=jnp.float32)
    h = jnp.maximum(h + cbig_ref[...], 0.0)
    ps = jnp.sum(h, axis=1)
    pooled = jnp.sum(ps.reshape(conv_mid, 8), axis=1, keepdims=True) * inv_hw
    feat = jnp.sum(pooled * fw_ref[...], axis=0, keepdims=True) + fb_ref[...]
    o_ref[0] = jnp.maximum(feat, 0.0).astype(jnp.bfloat16)


def _backbone_forward(batch_img, conv_wt, conv_b, bk_fc_w, bk_fc_b):
    B, C, H, W = batch_img.shape
    HW = H * W
    conv_mid = conv_wt.shape[0]
    feat_dim = bk_fc_w.shape[1]
    lanes = C * HW // 24
    x24 = batch_img.reshape(B, 24, lanes)
    w_big = jnp.kron(conv_wt, jnp.eye(8, dtype=conv_wt.dtype))
    cb_big = jnp.kron(conv_b, jnp.ones((8, 1), conv_b.dtype))

    flops = 2 * B * HW * C * conv_mid + 2 * B * conv_mid * feat_dim
    bytes_acc = (B * C * HW * 4 + C * conv_mid * 2 + conv_mid * feat_dim * 4
                 + B * feat_dim * 2)
    out = pl.pallas_call(
        functools.partial(_backbone_kernel, conv_mid=conv_mid,
                          inv_hw=1.0 / float(HW)),
        out_shape=jax.ShapeDtypeStruct((B, 1, feat_dim), jnp.bfloat16),
        grid_spec=pltpu.PrefetchScalarGridSpec(
            num_scalar_prefetch=0,
            grid=(B,),
            in_specs=[
                pl.BlockSpec((1, 24, lanes), lambda b: (b, 0, 0)),
                pl.BlockSpec((8 * conv_mid, 24), lambda b: (0, 0)),
                pl.BlockSpec((8 * conv_mid, 1), lambda b: (0, 0)),
                pl.BlockSpec((conv_mid, feat_dim), lambda b: (0, 0)),
                pl.BlockSpec((1, feat_dim), lambda b: (0, 0)),
            ],
            out_specs=pl.BlockSpec((1, 1, feat_dim), lambda b: (b, 0, 0)),
        ),
        compiler_params=pltpu.CompilerParams(
            dimension_semantics=("parallel",),
            vmem_limit_bytes=64 * 1024 * 1024),
        cost_estimate=pl.CostEstimate(flops=flops, transcendentals=0,
                                      bytes_accessed=bytes_acc),
    )(x24, w_big, cb_big, bk_fc_w, bk_fc_b)
    return out.reshape(B, feat_dim)


def _gcn_kernel(u_ref, fnw1_ref, fnb1_ref, fnw2_ref, fnb2_ref,
                a_ref, w1_ref, b1_ref, w2_ref, b2_ref, w3_ref, b3_ref,
                r_ref, o_ref, *, num_graphs, node_size):
    GN = num_graphs * node_size

    u = u_ref[...].astype(jnp.bfloat16)
    t = jnp.dot(u, fnw1_ref[...], preferred_element_type=jnp.float32) + fnb1_ref[...]
    t = jnp.maximum(t, 0.0).astype(jnp.bfloat16)
    h0 = jnp.dot(t, fnw2_ref[...], preferred_element_type=jnp.float32) + fnb2_ref[...]

    A = a_ref[...]

    h0b = h0.astype(jnp.bfloat16)
    tiled = jnp.concatenate([h0b] * num_graphs, axis=1)
    row_g = lax.broadcasted_iota(jnp.int32, (GN, GN), 0) // node_size
    col_g = lax.broadcasted_iota(jnp.int32, (GN, GN), 1) // node_size
    ht = jnp.where(row_g == col_g, tiled, jnp.bfloat16(0))
    z = jnp.dot(ht, w1_ref[...], preferred_element_type=jnp.float32)
    z = jnp.dot(A, z.astype(jnp.bfloat16),
                preferred_element_type=jnp.float32) + b1_ref[...]
    h1 = jnp.maximum(z, 0.0).astype(jnp.bfloat16)

    z = jnp.dot(h1, w2_ref[...], preferred_element_type=jnp.float32)
    z = jnp.dot(A, z.astype(jnp.bfloat16),
                preferred_element_type=jnp.float32) + b2_ref[...]
    h2 = jnp.maximum(z, 0.0).astype(jnp.bfloat16)

    z = jnp.dot(h2, w3_ref[...], preferred_element_type=jnp.float32)
    z = jnp.dot(A, z.astype(jnp.bfloat16),
                preferred_element_type=jnp.float32) + b3_ref[...]
    o_ref[...] = jnp.dot(r_ref[...], z.astype(jnp.bfloat16),
                         preferred_element_type=jnp.float32).astype(jnp.bfloat16)


def _gcn_forward(batch_u, A_bd, R, fn_w1, fn_b1, fn_w2, fn_b2,
                 gcn_w1, gcn_b1, gcn_w2, gcn_b2, gcn_w3, gcn_b3):
    G, Nn, Fin = batch_u.shape
    GN = G * Nn
    H256 = fn_w1.shape[1]
    ns = fn_w2.shape[1]
    H1 = gcn_w1.shape[1]
    H2 = gcn_w2.shape[1]
    Fo = gcn_w3.shape[1]
    u2d = batch_u.reshape(GN, Fin)

    flops = (2 * GN * (Fin * H256 + H256 * ns) + 2 * GN * GN * H1
             + 2 * GN * GN * (H1 + H2 + Fo)
             + 2 * GN * (H1 * H2 + H2 * Fo) + 2 * G * GN * Fo)
    bytes_acc = (GN * Fin * 4 + Fin * H256 * 2 + H256 * ns * 2 + GN * H1 * 2
                 + H1 * H2 * 2 + H2 * Fo * 2 + GN * GN * 2 + G * GN * 2
                 + G * Fo * 2)

    return pl.pallas_call(
        functools.partial(_gcn_kernel, num_graphs=G, node_size=Nn),
        out_shape=jax.ShapeDtypeStruct((G, Fo), jnp.bfloat16),
        grid_spec=pltpu.PrefetchScalarGridSpec(
            num_scalar_prefetch=0,
            grid=(1,),
            in_specs=[
                pl.BlockSpec((GN, Fin), lambda i: (0, 0)),
                pl.BlockSpec((Fin, H256), lambda i: (0, 0)),
                pl.BlockSpec((1, H256), lambda i: (0, 0)),
                pl.BlockSpec((H256, ns), lambda i: (0, 0)),
                pl.BlockSpec((1, ns), lambda i: (0, 0)),
                pl.BlockSpec((GN, GN), lambda i: (0, 0)),
                pl.BlockSpec((GN, H1), lambda i: (0, 0)),
                pl.BlockSpec((1, H1), lambda i: (0, 0)),
                pl.BlockSpec((H1, H2), lambda i: (0, 0)),
                pl.BlockSpec((1, H2), lambda i: (0, 0)),
                pl.BlockSpec((H2, Fo), lambda i: (0, 0)),
                pl.BlockSpec((1, Fo), lambda i: (0, 0)),
                pl.BlockSpec((G, GN), lambda i: (0, 0)),
            ],
            out_specs=pl.BlockSpec((G, Fo), lambda i: (0, 0)),
        ),
        compiler_params=pltpu.CompilerParams(
            dimension_semantics=("arbitrary",)),
        cost_estimate=pl.CostEstimate(flops=flops, transcendentals=0,
                                      bytes_accessed=bytes_acc),
    )(u2d, fn_w1, fn_b1, fn_w2, fn_b2, A_bd, gcn_w1, gcn_b1,
      gcn_w2, gcn_b2, gcn_w3, gcn_b3, R)


def _classifier_kernel(f_ref, fg_ref, w1a_ref, w1b_ref, b1_ref, w2_ref, b2_ref,
                       o_ref, f_sc, fg_sc):
    j = pl.program_id(0)

    @pl.when(j == 0)
    def _():
        o_ref[...] = jnp.zeros_like(o_ref)
        f_sc[...] = jnp.maximum(f_ref[...], 0.0).astype(jnp.bfloat16)
        fg_sc[...] = jnp.maximum(fg_ref[...], 0.0).astype(jnp.bfloat16)

    h = jnp.dot(f_sc[...], w1a_ref[...], preferred_element_type=jnp.float32)
    h = h + jnp.dot(fg_sc[...], w1b_ref[...], preferred_element_type=jnp.float32)
    h = jnp.maximum(h + b1_ref[...], 0.0).astype(jnp.bfloat16)
    o_ref[...] += jnp.dot(h, w2_ref[...], preferred_element_type=jnp.float32)

    @pl.when(j == pl.num_programs(0) - 1)
    def _():
        o_ref[...] = o_ref[...] + b2_ref[...]


def _classifier_forward(feature, feature_g, fc_w1a, fc_b1, fc_w1b, fc_w2, fc_b2):
    B, F1 = feature.shape
    G, F2 = feature_g.shape
    H1 = fc_w1a.shape[1]
    num_classes = fc_w2.shape[1]

    t_n1 = _pick_lane_tile(H1, 512)
    n1 = H1 // t_n1
    Np = _round_up(max(num_classes, 128), 128)
    w2p = jnp.pad(fc_w2, ((0, 0), (0, Np - num_classes)))
    b2p = jnp.pad(fc_b2, ((0, 0), (0, Np - num_classes)))

    flops = 2 * B * (F1 * H1 + F2 * H1 + H1 * Np)
    bytes_acc = (F1 * H1 * 2 + F2 * H1 * 2 + H1 * Np * 2
                 + B * (F1 + F2) * 2 + B * Np * 4)
    out = pl.pallas_call(
        _classifier_kernel,
        out_shape=jax.ShapeDtypeStruct((B, Np), jnp.float32),
        grid_spec=pltpu.PrefetchScalarGridSpec(
            num_scalar_prefetch=0,
            grid=(n1,),
            in_specs=[
                pl.BlockSpec((B, F1), lambda j: (0, 0)),
                pl.BlockSpec((B, F2), lambda j: (0, 0)),
                pl.BlockSpec((F1, t_n1), lambda j: (0, j)),
                pl.BlockSpec((F2, t_n1), lambda j: (0, j)),
                pl.BlockSpec((1, t_n1), lambda j: (0, j)),
                pl.BlockSpec((t_n1, Np), lambda j: (j, 0)),
                pl.BlockSpec((1, Np), lambda j: (0, 0)),
            ],
            out_specs=pl.BlockSpec((B, Np), lambda j: (0, 0)),
            scratch_shapes=[pltpu.VMEM((B, F1), jnp.bfloat16),
                            pltpu.VMEM((B, F2), jnp.bfloat16)],
        ),
        compiler_params=pltpu.CompilerParams(
            dimension_semantics=("arbitrary",),
            vmem_limit_bytes=64 * 1024 * 1024),
        cost_estimate=pl.CostEstimate(flops=flops, transcendentals=0,
                                      bytes_accessed=bytes_acc),
    )(feature, feature_g, fc_w1a, fc_w1b, fc_b1, w2p, b2p)
    return out[:, :num_classes]


def _build_norm_adj(node_num):
    idx = jnp.arange(node_num)
    A = jnp.zeros((node_num, node_num), jnp.float32)
    A = A.at[idx, (idx + 1) % node_num].set(1.0)
    A = A.at[(idx + 1) % node_num, idx].set(1.0)
    A = A + jnp.eye(node_num, dtype=jnp.float32)
    dinv = 1.0 / jnp.sqrt(A.sum(axis=1))
    return A * dinv[:, None] * dinv[None, :]


def kernel(batch_img, batch_u, conv_wt, conv_b, bk_fc_w, bk_fc_b,
           fn_w1, fn_b1, fn_w2, fn_b2, gcn_w1, gcn_b1, gcn_w2, gcn_b2,
           gcn_w3, gcn_b3, fc_w1a, fc_b1, fc_w1b, fc_w2, fc_b2):
    B, C, H, W = batch_img.shape
    HW = H * W
    G, node_num, _ = batch_u.shape

    feature = _backbone_forward(batch_img, conv_wt, conv_b, bk_fc_w, bk_fc_b)
    A_hat = _build_norm_adj(node_num)
    A_bd = jnp.kron(jnp.eye(G, dtype=jnp.float32), A_hat).astype(jnp.bfloat16)
    R = jnp.kron(jnp.eye(G, dtype=jnp.float32),
                 jnp.full((1, node_num), 1.0 / node_num, jnp.float32)
                 ).astype(jnp.bfloat16)

    feature_g = _gcn_forward(batch_u, A_bd, R, fn_w1, fn_b1, fn_w2, fn_b2,
                             gcn_w1, gcn_b1, gcn_w2, gcn_b2, gcn_w3, gcn_b3)

    return _classifier_forward(feature, feature_g,
                               fc_w1a, fc_b1, fc_w1b, fc_w2, fc_b2)


# speedup vs baseline: 4.5230x; 1.0017x over previous
import functools

import jax
import jax.numpy as jnp
from jax import lax
from jax.experimental import pallas as pl
from jax.experimental.pallas import tpu as pltpu


def _round_up(x, m):
    return ((x + m - 1) // m) * m


def _pick_lane_tile(n, cap):
    best = None
    t = 128
    while t <= min(n, cap):
        if n % t == 0:
            best = t
        t += 128
    return best if best is not None else n


def _fused_kernel(x_ref, wbig_ref, cbig_ref, fw_ref, fb_ref,
                  u_ref, fnw1_ref, fnb1_ref, fnw2_ref, fnb2_ref,
                  a_ref, w1_ref, b1_ref, w2_ref, b2_ref, w3_ref, b3_ref,
                  r_ref, w1a_ref, w1b_ref, cb1_ref, cw2_ref, cb2_ref,
                  o_ref, feat_sc, fg_sc,
                  *, n_img_steps, n1, num_graphs, node_size, conv_mid, inv_hw):
    s = pl.program_id(0)

    @pl.when(s == 0)
    def _init():
        feat_sc[...] = jnp.zeros_like(feat_sc)

    @pl.when(s < n_img_steps)
    def _backbone():
        x = x_ref[0].astype(jnp.bfloat16)
        h = jnp.dot(wbig_ref[...], x, preferred_element_type=jnp.float32)
        h = jnp.maximum(h + cbig_ref[...], 0.0)
        ps = jnp.sum(h, axis=1)
        pooled = jnp.sum(ps.reshape(2, conv_mid, 8), axis=2) * inv_hw
        feat = jnp.dot(pooled.astype(jnp.bfloat16), fw_ref[...],
                       preferred_element_type=jnp.float32) + fb_ref[...]
        featb = jnp.maximum(feat, 0.0).astype(jnp.bfloat16)
        rows_i = lax.broadcasted_iota(jnp.int32, (feat_sc.shape[0], 1), 0)
        m0 = (rows_i == 2 * s).astype(jnp.bfloat16)
        m1 = (rows_i == 2 * s + 1).astype(jnp.bfloat16)
        feat_sc[...] += m0 * featb[0:1, :] + m1 * featb[1:2, :]

    @pl.when(s == n_img_steps)
    def _gcn():
        GN = num_graphs * node_size
        u = u_ref[...].astype(jnp.bfloat16)
        t = jnp.dot(u, fnw1_ref[...],
                    preferred_element_type=jnp.float32) + fnb1_ref[...]
        t = jnp.maximum(t, 0.0).astype(jnp.bfloat16)
        h0 = jnp.dot(t, fnw2_ref[...],
                     preferred_element_type=jnp.float32) + fnb2_ref[...]

        A = a_ref[...]
        h0b = h0.astype(jnp.bfloat16)
        tiled = jnp.concatenate([h0b] * num_graphs, axis=1)
        row_g = lax.broadcasted_iota(jnp.int32, (GN, GN), 0) // node_size
        col_g = lax.broadcasted_iota(jnp.int32, (GN, GN), 1) // node_size
        ht = jnp.where(row_g == col_g, tiled, jnp.bfloat16(0))
        z = jnp.dot(ht, w1_ref[...], preferred_element_type=jnp.float32)
        z = jnp.dot(A, z.astype(jnp.bfloat16),
                    preferred_element_type=jnp.float32) + b1_ref[...]
        h1 = jnp.maximum(z, 0.0).astype(jnp.bfloat16)

        z = jnp.dot(h1, w2_ref[...], preferred_element_type=jnp.float32)
        z = jnp.dot(A, z.astype(jnp.bfloat16),
                    preferred_element_type=jnp.float32) + b2_ref[...]
        h2 = jnp.maximum(z, 0.0).astype(jnp.bfloat16)

        z = jnp.dot(h2, w3_ref[...], preferred_element_type=jnp.float32)
        z = jnp.dot(A, z.astype(jnp.bfloat16),
                    preferred_element_type=jnp.float32) + b3_ref[...]
        fg = jnp.dot(r_ref[...], z.astype(jnp.bfloat16),
                     preferred_element_type=jnp.float32)
        fg_sc[...] = jnp.maximum(fg, 0.0).astype(jnp.bfloat16)

    @pl.when(s > n_img_steps)
    def _classifier():
        j = s - (n_img_steps + 1)

        @pl.when(j == 0)
        def _():
            o_ref[...] = jnp.zeros_like(o_ref)

        h = jnp.dot(feat_sc[...], w1a_ref[...],
                    preferred_element_type=jnp.float32)
        h = h + jnp.dot(fg_sc[...], w1b_ref[...],
                        preferred_element_type=jnp.float32)
        h = jnp.maximum(h + cb1_ref[...], 0.0).astype(jnp.bfloat16)
        o_ref[...] += jnp.dot(h, cw2_ref[...],
                              preferred_element_type=jnp.float32)

        @pl.when(j == n1 - 1)
        def _():
            o_ref[...] = o_ref[...] + cb2_ref[...]


def _build_norm_adj(node_num):
    idx = jnp.arange(node_num)
    A = jnp.zeros((node_num, node_num), jnp.float32)
    A = A.at[idx, (idx + 1) % node_num].set(1.0)
    A = A.at[(idx + 1) % node_num, idx].set(1.0)
    A = A + jnp.eye(node_num, dtype=jnp.float32)
    dinv = 1.0 / jnp.sqrt(A.sum(axis=1))
    return A * dinv[:, None] * dinv[None, :]


def kernel(batch_img, batch_u, conv_wt, conv_b, bk_fc_w, bk_fc_b,
           fn_w1, fn_b1, fn_w2, fn_b2, gcn_w1, gcn_b1, gcn_w2, gcn_b2,
           gcn_w3, gcn_b3, fc_w1a, fc_b1, fc_w1b, fc_w2, fc_b2):
    B, C, H, W = batch_img.shape
    HW = H * W
    G, node_num, Fin = batch_u.shape
    GN = G * node_num
    conv_mid = conv_wt.shape[0]
    feat_dim = bk_fc_w.shape[1]
    H256 = fn_w1.shape[1]
    ns = fn_w2.shape[1]
    H1 = gcn_w1.shape[1]
    H2 = gcn_w2.shape[1]
    Fo = gcn_w3.shape[1]
    F1 = fc_w1a.shape[0]
    F2 = fc_w1b.shape[0]
    HC = fc_w1a.shape[1]
    num_classes = fc_w2.shape[1]

    ipp = 2
    rows = 24 * ipp
    lanes = C * HW // 24
    n_img_steps = B // ipp
    x48 = batch_img.reshape(n_img_steps, rows, lanes)
    w_big = jnp.kron(jnp.eye(ipp, dtype=conv_wt.dtype),
                     jnp.kron(conv_wt, jnp.eye(8, dtype=conv_wt.dtype)))
    cb_big = jnp.kron(jnp.ones((ipp, 1), conv_b.dtype),
                      jnp.kron(conv_b, jnp.ones((8, 1), conv_b.dtype)))
    fw_bf = bk_fc_w.astype(jnp.bfloat16)

    A_hat = _build_norm_adj(node_num)
    A_bd = jnp.kron(jnp.eye(G, dtype=jnp.float32), A_hat).astype(jnp.bfloat16)
    R = jnp.kron(jnp.eye(G, dtype=jnp.float32),
                 jnp.full((1, node_num), 1.0 / node_num, jnp.float32)
                 ).astype(jnp.bfloat16)
    u2d = batch_u.reshape(GN, Fin)

    t_n1 = _pick_lane_tile(HC, 512)
    n1 = HC // t_n1
    Np = _round_up(max(num_classes, 128), 128)
    w2p = jnp.pad(fc_w2, ((0, 0), (0, Np - num_classes)))
    b2p = jnp.pad(fc_b2, ((0, 0), (0, Np - num_classes)))

    n_steps = n_img_steps + 1 + n1
    gi = n_img_steps

    def img_idx(sv):
        return (jnp.minimum(sv, n_img_steps - 1), 0, 0)

    def cls_idx_col(sv):
        return (0, jnp.clip(sv - (gi + 1), 0, n1 - 1))

    def cls_idx_row(sv):
        return (jnp.clip(sv - (gi + 1), 0, n1 - 1), 0)

    const2 = lambda sv: (0, 0)
    const3 = lambda sv: (0, 0, 0)

    flops = (2 * B * HW * C * conv_mid + 2 * B * conv_mid * feat_dim
             + 2 * GN * (Fin * H256 + H256 * ns) + 2 * GN * GN * H1
             + 2 * GN * GN * (H1 + H2 + Fo) + 2 * GN * (H1 * H2 + H2 * Fo)
             + 2 * G * GN * Fo + 2 * B * (F1 * HC + F2 * HC + HC * Np))
    bytes_acc = (B * C * HW * 4 + conv_mid * feat_dim * 2
                 + GN * Fin * 4 + GN * GN * 2 + GN * (H1 + H2) * 2
                 + H2 * Fo * 2 + (F1 + F2) * HC * 2 + HC * Np * 2
                 + B * Np * 4)

    out = pl.pallas_call(
        functools.partial(_fused_kernel, n_img_steps=n_img_steps, n1=n1,
                          num_graphs=G, node_size=node_num,
                          conv_mid=conv_mid, inv_hw=1.0 / float(HW)),
        out_shape=jax.ShapeDtypeStruct((B, Np), jnp.float32),
        grid_spec=pltpu.PrefetchScalarGridSpec(
            num_scalar_prefetch=0,
            grid=(n_steps,),
            in_specs=[
                pl.BlockSpec((1, rows, lanes), img_idx),
                pl.BlockSpec((ipp * 8 * conv_mid, rows), const2),
                pl.BlockSpec((ipp * 8 * conv_mid, 1), const2),
                pl.BlockSpec((conv_mid, feat_dim), const2),
                pl.BlockSpec((1, feat_dim), const2),
                pl.BlockSpec((GN, Fin), const2),
                pl.BlockSpec((Fin, H256), const2),
                pl.BlockSpec((1, H256), const2),
                pl.BlockSpec((H256, ns), const2),
                pl.BlockSpec((1, ns), const2),
                pl.BlockSpec((GN, GN), const2),
                pl.BlockSpec((GN, H1), const2),
                pl.BlockSpec((1, H1), const2),
                pl.BlockSpec((H1, H2), const2),
                pl.BlockSpec((1, H2), const2),
                pl.BlockSpec((H2, Fo), const2),
                pl.BlockSpec((1, Fo), const2),
                pl.BlockSpec((G, GN), const2),
                pl.BlockSpec((F1, t_n1), cls_idx_col),
                pl.BlockSpec((F2, t_n1), cls_idx_col),
                pl.BlockSpec((1, t_n1), cls_idx_col),
                pl.BlockSpec((t_n1, Np), cls_idx_row),
                pl.BlockSpec((1, Np), const2),
            ],
            out_specs=pl.BlockSpec((B, Np), const2),
            scratch_shapes=[pltpu.VMEM((B, feat_dim), jnp.bfloat16),
                            pltpu.VMEM((G, Fo), jnp.bfloat16)],
        ),
        compiler_params=pltpu.CompilerParams(
            dimension_semantics=("arbitrary",),
            vmem_limit_bytes=100 * 1024 * 1024),
        cost_estimate=pl.CostEstimate(flops=flops, transcendentals=0,
                                      bytes_accessed=bytes_acc),
    )(x48, w_big, cb_big, fw_bf, bk_fc_b,
      u2d, fn_w1, fn_b1, fn_w2, fn_b2,
      A_bd, gcn_w1, gcn_b1, gcn_w2, gcn_b2, gcn_w3, gcn_b3, R,
      fc_w1a, fc_w1b, fc_b1, w2p, b2p)
    return out[:, :num_classes]
